# baseline (speedup 1.0000x reference)
import functools

import jax
import jax.numpy as jnp
from jax import lax
from jax.experimental import pallas as pl
from jax.experimental.pallas import tpu as pltpu
from jax.experimental.pallas import tpu_sc as plsc

NUM_NODES = 1000000
EMBED_DIM = 64
BATCH = 16384

_info = plsc.get_sparse_core_info()
_NC, _NS = _info.num_cores, _info.num_subcores
_NW = _NC * _NS
_B_PER_W = BATCH // _NW
_SLAB_T = 640
_NT = NUM_NODES // 8
_SLABS_PER_SC = _NT // (_NC * _SLAB_T)

_mesh = plsc.VectorSubcoreMesh(core_axis_name="c", subcore_axis_name="s")


@functools.partial(
    pl.kernel,
    mesh=_mesh,
    out_type=jax.ShapeDtypeStruct((BATCH, EMBED_DIM), jnp.float32),
    scratch_types=[
        pltpu.VMEM_SHARED((2, _SLAB_T, 8, EMBED_DIM), jnp.float32),
        pltpu.VMEM((_B_PER_W, EMBED_DIM), jnp.float32),
        pltpu.SemaphoreType.DMA,
        pltpu.SemaphoreType.DMA,
    ],
)
def _probe_kernel(idx_hbm, table_hbm, out_hbm, slab_s, rows_v, sem_a, sem_b):
    cid = lax.axis_index("c")
    sid = lax.axis_index("s")
    wid = sid * _NC + cid
    base = wid * _B_PER_W
    table_view = table_hbm.reshape(_NT, 8, EMBED_DIM)
    sc_t0 = cid * (_NT // _NC)

    @pl.when(sid == 0)
    def _():
        def body(g, carry):
            lo = sc_t0 + g * _SLAB_T
            buf = lax.rem(g, 2)
            sem = sem_a

            @pl.when(g >= 2)
            def _():
                pltpu.make_async_copy(
                    table_view.at[pl.ds(0, _SLAB_T)],
                    slab_s.at[buf],
                    sem,
                ).wait()

            pltpu.make_async_copy(
                table_view.at[pl.ds(lo, _SLAB_T)],
                slab_s.at[buf],
                sem,
            ).start()
            return carry

        lax.fori_loop(0, _SLABS_PER_SC, body, 0)
        for _ in range(2):
            pltpu.make_async_copy(
                table_view.at[pl.ds(0, _SLAB_T)],
                slab_s.at[0],
                sem_a,
            ).wait()

    plsc.subcore_barrier()
    pltpu.sync_copy(rows_v, out_hbm.at[pl.ds(base, _B_PER_W)])


def kernel(indices, weight):
    idx = indices.astype(jnp.int32)
    return _probe_kernel(idx, weight)


# speedup vs baseline: 1.7765x; 1.7765x over previous
import functools

import jax
import jax.numpy as jnp
from jax import lax
from jax.experimental import pallas as pl
from jax.experimental.pallas import tpu as pltpu
from jax.experimental.pallas import tpu_sc as plsc

NUM_NODES = 1000000
EMBED_DIM = 64
BATCH = 16384

_info = plsc.get_sparse_core_info()
_NC, _NS = _info.num_cores, _info.num_subcores
_NW = _NC * _NS
_B_PER_W = BATCH // _NW

_mesh = plsc.VectorSubcoreMesh(core_axis_name="c", subcore_axis_name="s")


@functools.partial(
    pl.kernel,
    mesh=_mesh,
    out_type=jax.ShapeDtypeStruct((BATCH, EMBED_DIM), jnp.float32),
    scratch_types=[
        pltpu.VMEM((_B_PER_W,), jnp.int32),
        pltpu.VMEM((_B_PER_W, EMBED_DIM), jnp.float32),
        pltpu.SemaphoreType.DMA,
    ],
)
def _gather_kernel(idx_hbm, table_hbm, out_hbm, idx_v, rows_v, sem):
    wid = lax.axis_index("s") * _NC + lax.axis_index("c")
    base = wid * _B_PER_W
    pltpu.sync_copy(idx_hbm.at[pl.ds(base, _B_PER_W)], idx_v)

    def fire(g, carry):
        v = idx_v[pl.ds(g * 16, 16)]
        for l in range(16):
            pltpu.make_async_copy(
                table_hbm.at[pl.ds(v[l], 1)],
                rows_v.at[pl.ds(g * 16 + l, 1)],
                sem,
            ).start()
        return carry

    lax.fori_loop(0, _B_PER_W // 16, fire, 0)
    pltpu.make_async_copy(
        table_hbm.at[pl.ds(0, _B_PER_W)],
        rows_v,
        sem,
    ).wait()
    pltpu.sync_copy(rows_v, out_hbm.at[pl.ds(base, _B_PER_W)])


def kernel(indices, weight):
    idx = indices.astype(jnp.int32)
    return _gather_kernel(idx, weight)
